# Initial kernel scaffold; baseline (speedup 1.0000x reference)
import jax
import jax.numpy as jnp
from jax.experimental import pallas as pl


def kernel(patch, patch_lib):
    raise NotImplementedError("write your pallas kernel here")


# speedup vs baseline: 1.0000x; 1.0000x over previous
import functools

import jax
import jax.numpy as jnp
from jax.experimental import pallas as pl
from jax.experimental.pallas import tpu as pltpu


def _knn_body(q_ref, k_ref, q2_ref, k2_ref,
              minval_ref, minidx_ref, sstar_ref, sidx_ref,
              run_min, run_idx, *, kb_size, nk):
    kb = pl.program_id(0)
    q = q_ref[...]
    k = k_ref[...]
    qk = jax.lax.dot_general(
        q, k, dimension_numbers=(((1,), (1,)), ((), ())),
        preferred_element_type=jnp.float32)
    d2 = (q2_ref[...] + k2_ref[...]) - 2.0 * qk
    dist = jnp.sqrt(jnp.maximum(d2, 1e-12))

    m = jnp.min(dist, axis=1, keepdims=True)
    lanes = jax.lax.broadcasted_iota(jnp.int32, dist.shape, 1) + kb * kb_size
    big = jnp.int32(nk * kb_size)
    li = jnp.min(jnp.where(dist == m, lanes, big), axis=1, keepdims=True)

    @pl.when(kb == 0)
    def _():
        run_min[...] = m
        run_idx[...] = li

    @pl.when(kb > 0)
    def _():
        better = m < run_min[...]
        run_min[...] = jnp.where(better, m, run_min[...])
        run_idx[...] = jnp.where(better, li, run_idx[...])

    @pl.when(kb == nk - 1)
    def _():
        mv = run_min[...]
        minval_ref[...] = mv
        minidx_ref[...] = run_idx[...]
        s = jnp.max(mv, axis=0, keepdims=True)
        sstar_ref[...] = s
        rows = jax.lax.broadcasted_iota(jnp.int32, mv.shape, 0)
        q_n = mv.shape[0]
        sidx_ref[...] = jnp.min(jnp.where(mv == s, rows, jnp.int32(q_n)),
                                axis=0, keepdims=True)


def kernel(patch, patch_lib):
    q_n, d = patch.shape
    k_n, _ = patch_lib.shape
    kb_size = 2048
    nk = k_n // kb_size

    q2 = jnp.sum(patch * patch, axis=1, keepdims=True)
    k2 = jnp.sum(patch_lib * patch_lib, axis=1)[None, :]

    body = functools.partial(_knn_body, kb_size=kb_size, nk=nk)
    minval, minidx, sstar, sidx = pl.pallas_call(
        body,
        grid=(nk,),
        in_specs=[
            pl.BlockSpec((q_n, d), lambda i: (0, 0)),
            pl.BlockSpec((kb_size, d), lambda i: (i, 0)),
            pl.BlockSpec((q_n, 1), lambda i: (0, 0)),
            pl.BlockSpec((1, kb_size), lambda i: (0, i)),
        ],
        out_specs=[
            pl.BlockSpec((q_n, 1), lambda i: (0, 0)),
            pl.BlockSpec((q_n, 1), lambda i: (0, 0)),
            pl.BlockSpec((1, 1), lambda i: (0, 0)),
            pl.BlockSpec((1, 1), lambda i: (0, 0)),
        ],
        out_shape=[
            jax.ShapeDtypeStruct((q_n, 1), jnp.float32),
            jax.ShapeDtypeStruct((q_n, 1), jnp.int32),
            jax.ShapeDtypeStruct((1, 1), jnp.float32),
            jax.ShapeDtypeStruct((1, 1), jnp.int32),
        ],
        scratch_shapes=[
            pltpu.VMEM((q_n, 1), jnp.float32),
            pltpu.VMEM((q_n, 1), jnp.int32),
        ],
    )(patch, patch_lib, q2, k2)

    s_map = minval.reshape(1, 1, q_n)
    s_star = sstar.reshape(())
    s_idx = sidx.reshape(())
    min_idx = minidx.reshape(q_n)
    return (s_map, s_star, s_idx, min_idx)


# speedup vs baseline: 1.1157x; 1.1157x over previous
import functools

import jax
import jax.numpy as jnp
from jax.experimental import pallas as pl
from jax.experimental.pallas import tpu as pltpu


def _knn_body(qm2_ref, k_ref, q2_ref, k2_ref, lanes_ref,
              minval_ref, minidx_ref, sstar_ref, sidx_ref,
              run_min, run_idx, *, kb_size, nk):
    kb = pl.program_id(0)
    qm2 = qm2_ref[...]
    k = k_ref[...]
    qk = jax.lax.dot_general(
        qm2, k, dimension_numbers=(((1,), (1,)), ((), ())),
        preferred_element_type=jnp.float32)
    d2 = (q2_ref[...] + k2_ref[...]) + qk
    dist = jnp.sqrt(jnp.maximum(d2, 1e-12))

    m = jnp.min(dist, axis=1, keepdims=True)
    lanes = lanes_ref[...]
    big = jnp.float32(kb_size)
    li = (jnp.min(jnp.where(dist == m, lanes, big), axis=1, keepdims=True)
          + (kb * kb_size).astype(jnp.float32))

    @pl.when(kb == 0)
    def _():
        run_min[...] = m
        run_idx[...] = li

    @pl.when(kb > 0)
    def _():
        better = m < run_min[...]
        run_min[...] = jnp.where(better, m, run_min[...])
        run_idx[...] = jnp.where(better, li, run_idx[...])

    @pl.when(kb == nk - 1)
    def _():
        mv = run_min[...]
        minval_ref[...] = mv
        minidx_ref[...] = run_idx[...].astype(jnp.int32)
        s = jnp.max(mv, axis=0, keepdims=True)
        sstar_ref[...] = s
        rows = jax.lax.broadcasted_iota(jnp.int32, mv.shape, 0).astype(jnp.float32)
        q_n = mv.shape[0]
        sidx_ref[...] = jnp.min(jnp.where(mv == s, rows, jnp.float32(q_n)),
                                axis=0, keepdims=True).astype(jnp.int32)


def kernel(patch, patch_lib):
    q_n, d = patch.shape
    k_n, _ = patch_lib.shape
    kb_size = 2048
    nk = k_n // kb_size

    q2 = jnp.sum(patch * patch, axis=1, keepdims=True)
    k2 = jnp.sum(patch_lib * patch_lib, axis=1)[None, :]
    qm2 = patch * jnp.float32(-2.0)
    lanes = jnp.arange(kb_size, dtype=jnp.float32)[None, :]

    body = functools.partial(_knn_body, kb_size=kb_size, nk=nk)
    minval, minidx, sstar, sidx = pl.pallas_call(
        body,
        grid=(nk,),
        in_specs=[
            pl.BlockSpec((q_n, d), lambda i: (0, 0)),
            pl.BlockSpec((kb_size, d), lambda i: (i, 0)),
            pl.BlockSpec((q_n, 1), lambda i: (0, 0)),
            pl.BlockSpec((1, kb_size), lambda i: (0, i)),
            pl.BlockSpec((1, kb_size), lambda i: (0, 0)),
        ],
        out_specs=[
            pl.BlockSpec((q_n, 1), lambda i: (0, 0)),
            pl.BlockSpec((q_n, 1), lambda i: (0, 0)),
            pl.BlockSpec((1, 1), lambda i: (0, 0)),
            pl.BlockSpec((1, 1), lambda i: (0, 0)),
        ],
        out_shape=[
            jax.ShapeDtypeStruct((q_n, 1), jnp.float32),
            jax.ShapeDtypeStruct((q_n, 1), jnp.int32),
            jax.ShapeDtypeStruct((1, 1), jnp.float32),
            jax.ShapeDtypeStruct((1, 1), jnp.int32),
        ],
        scratch_shapes=[
            pltpu.VMEM((q_n, 1), jnp.float32),
            pltpu.VMEM((q_n, 1), jnp.float32),
        ],
    )(qm2, patch_lib, q2, k2, lanes)

    s_map = minval.reshape(1, 1, q_n)
    s_star = sstar.reshape(())
    s_idx = sidx.reshape(())
    min_idx = minidx.reshape(q_n)
    return (s_map, s_star, s_idx, min_idx)


# speedup vs baseline: 1.5112x; 1.3545x over previous
import functools

import jax
import jax.numpy as jnp
from jax.experimental import pallas as pl
from jax.experimental.pallas import tpu as pltpu


def _knn_body(qm2_ref, k_ref, q2_ref, k2_ref, ridx_ref, qiota_ref,
              minval_ref, minidx_ref, sstar_ref, sidx_ref,
              run_min, run_idx, *, kb_size, nk):
    kb = pl.program_id(0)
    k = k_ref[...]
    qm2 = qm2_ref[...]
    qk = jax.lax.dot_general(
        k, qm2, dimension_numbers=(((1,), (1,)), ((), ())),
        preferred_element_type=jnp.float32)
    d2 = (q2_ref[...] + k2_ref[...]) + qk

    m2 = jnp.min(d2, axis=0, keepdims=True)
    cm = jnp.maximum(m2, 1e-12)
    s = jnp.sqrt(cm)
    yi = jax.lax.bitcast_convert_type(s * s, jnp.int32)
    h = cm
    for step in range(6):
        yk = jax.lax.bitcast_convert_type(yi + step, jnp.float32)
        h = jnp.maximum(h, jnp.where(jnp.sqrt(yk) == s, yk, cm))

    ridx = ridx_ref[...]
    big = jnp.float32(kb_size)
    li = (jnp.min(jnp.where(d2 <= h, ridx, big), axis=0, keepdims=True)
          + (kb * kb_size).astype(jnp.float32))

    @pl.when(kb == 0)
    def _():
        run_min[...] = s
        run_idx[...] = li

    @pl.when(kb > 0)
    def _():
        better = s < run_min[...]
        run_min[...] = jnp.where(better, s, run_min[...])
        run_idx[...] = jnp.where(better, li, run_idx[...])

    @pl.when(kb == nk - 1)
    def _():
        mv = run_min[...]
        minval_ref[...] = mv
        minidx_ref[...] = run_idx[...].astype(jnp.int32)
        st = jnp.max(mv, axis=1, keepdims=True)
        sstar_ref[...] = st
        q_n = mv.shape[1]
        sidx_ref[...] = jnp.min(
            jnp.where(mv == st, qiota_ref[...], jnp.float32(q_n)),
            axis=1, keepdims=True).astype(jnp.int32)


def kernel(patch, patch_lib):
    q_n, d = patch.shape
    k_n, _ = patch_lib.shape
    kb_size = 2048
    nk = k_n // kb_size

    q2 = jnp.sum(patch * patch, axis=1)[None, :]
    k2 = jnp.sum(patch_lib * patch_lib, axis=1)[:, None]
    qm2 = patch * jnp.float32(-2.0)
    ridx = jnp.arange(kb_size, dtype=jnp.float32)[:, None]
    qiota = jnp.arange(q_n, dtype=jnp.float32)[None, :]

    body = functools.partial(_knn_body, kb_size=kb_size, nk=nk)
    minval, minidx, sstar, sidx = pl.pallas_call(
        body,
        grid=(nk,),
        in_specs=[
            pl.BlockSpec((q_n, d), lambda i: (0, 0)),
            pl.BlockSpec((kb_size, d), lambda i: (i, 0)),
            pl.BlockSpec((1, q_n), lambda i: (0, 0)),
            pl.BlockSpec((kb_size, 1), lambda i: (i, 0)),
            pl.BlockSpec((kb_size, 1), lambda i: (0, 0)),
            pl.BlockSpec((1, q_n), lambda i: (0, 0)),
        ],
        out_specs=[
            pl.BlockSpec((1, q_n), lambda i: (0, 0)),
            pl.BlockSpec((1, q_n), lambda i: (0, 0)),
            pl.BlockSpec((1, 1), lambda i: (0, 0)),
            pl.BlockSpec((1, 1), lambda i: (0, 0)),
        ],
        out_shape=[
            jax.ShapeDtypeStruct((1, q_n), jnp.float32),
            jax.ShapeDtypeStruct((1, q_n), jnp.int32),
            jax.ShapeDtypeStruct((1, 1), jnp.float32),
            jax.ShapeDtypeStruct((1, 1), jnp.int32),
        ],
        scratch_shapes=[
            pltpu.VMEM((1, q_n), jnp.float32),
            pltpu.VMEM((1, q_n), jnp.float32),
        ],
    )(qm2, patch_lib, q2, k2, ridx, qiota)

    s_map = minval.reshape(1, 1, q_n)
    s_star = sstar.reshape(())
    s_idx = sidx.reshape(())
    min_idx = minidx.reshape(q_n)
    return (s_map, s_star, s_idx, min_idx)


# speedup vs baseline: 1.6763x; 1.1092x over previous
import functools

import jax
import jax.numpy as jnp
from jax.experimental import pallas as pl
from jax.experimental.pallas import tpu as pltpu


def _knn_body(q_ref, k_ref, q2_ref, k2_ref,
              minval_ref, minidx_ref, sstar_ref, sidx_ref,
              qm2_s, ridx_s, run_min, run_idx, *, kb_size, nk):
    kb = pl.program_id(0)

    @pl.when(kb == 0)
    def _():
        qm2_s[...] = q_ref[...] * jnp.float32(-2.0)
        ridx_s[...] = jax.lax.broadcasted_iota(
            jnp.int32, (kb_size, 1), 0).astype(jnp.float32)

    k = k_ref[...]
    qk = jax.lax.dot_general(
        k, qm2_s[...], dimension_numbers=(((1,), (1,)), ((), ())),
        preferred_element_type=jnp.float32)
    d2 = (q2_ref[...] + k2_ref[...]) + qk

    m2 = jnp.min(d2, axis=0, keepdims=True)
    cm = jnp.maximum(m2, 1e-12)
    s = jnp.sqrt(cm)
    yi = jax.lax.bitcast_convert_type(s * s, jnp.int32)
    h = cm
    for step in range(6):
        yk = jax.lax.bitcast_convert_type(yi + step, jnp.float32)
        h = jnp.maximum(h, jnp.where(jnp.sqrt(yk) == s, yk, cm))

    big = jnp.float32(kb_size)
    li = (jnp.min(jnp.where(d2 <= h, ridx_s[...], big), axis=0, keepdims=True)
          + (kb * kb_size).astype(jnp.float32))

    @pl.when(kb == 0)
    def _():
        run_min[...] = s
        run_idx[...] = li

    @pl.when(kb > 0)
    def _():
        better = s < run_min[...]
        run_min[...] = jnp.where(better, s, run_min[...])
        run_idx[...] = jnp.where(better, li, run_idx[...])

    @pl.when(kb == nk - 1)
    def _():
        mv = run_min[...]
        q_n = mv.shape[1]
        minval_ref[...] = mv.reshape(1, 1, q_n)
        minidx_ref[...] = run_idx[...].astype(jnp.int32).reshape(q_n)
        st = jnp.max(mv, axis=1, keepdims=True)
        sstar_ref[...] = st
        qiota = jax.lax.broadcasted_iota(
            jnp.int32, mv.shape, 1).astype(jnp.float32)
        sidx_ref[...] = jnp.min(
            jnp.where(mv == st, qiota, jnp.float32(q_n)),
            axis=1, keepdims=True).astype(jnp.int32)


def kernel(patch, patch_lib):
    q_n, d = patch.shape
    k_n, _ = patch_lib.shape
    kb_size = 2048
    nk = k_n // kb_size

    q2 = jnp.sum(patch * patch, axis=1)[None, :]
    k2 = jnp.sum(patch_lib * patch_lib, axis=1)[:, None]

    body = functools.partial(_knn_body, kb_size=kb_size, nk=nk)
    s_map, min_idx, sstar, sidx = pl.pallas_call(
        body,
        grid=(nk,),
        in_specs=[
            pl.BlockSpec((q_n, d), lambda i: (0, 0)),
            pl.BlockSpec((kb_size, d), lambda i: (i, 0)),
            pl.BlockSpec((1, q_n), lambda i: (0, 0)),
            pl.BlockSpec((kb_size, 1), lambda i: (i, 0)),
        ],
        out_specs=[
            pl.BlockSpec((1, 1, q_n), lambda i: (0, 0, 0)),
            pl.BlockSpec((q_n,), lambda i: (0,)),
            pl.BlockSpec((1, 1), lambda i: (0, 0)),
            pl.BlockSpec((1, 1), lambda i: (0, 0)),
        ],
        out_shape=[
            jax.ShapeDtypeStruct((1, 1, q_n), jnp.float32),
            jax.ShapeDtypeStruct((q_n,), jnp.int32),
            jax.ShapeDtypeStruct((1, 1), jnp.float32),
            jax.ShapeDtypeStruct((1, 1), jnp.int32),
        ],
        scratch_shapes=[
            pltpu.VMEM((q_n, d), jnp.float32),
            pltpu.VMEM((kb_size, 1), jnp.float32),
            pltpu.VMEM((1, q_n), jnp.float32),
            pltpu.VMEM((1, q_n), jnp.float32),
        ],
    )(patch, patch_lib, q2, k2)

    s_star = sstar.reshape(())
    s_idx = sidx.reshape(())
    return (s_map, s_star, s_idx, min_idx)
